# baseline (speedup 1.0000x reference)
import jax
import jax.numpy as jnp
from jax.experimental import pallas as pl
from jax.experimental.pallas import tpu as pltpu

BATCH = 4096
N_NODES = 8
IN_F = 256
OUT_F = 256
BM = 256


def _gcn_tile(adj_ref, x_ref, w_ref, b_ref, o_ref):
    w = w_ref[...]
    b = b_ref[...]
    x = x_ref[...]
    xs = [x[:, m, :] for m in range(N_NODES)]
    for n in range(N_NODES):
        acc = xs[0] * adj_ref[n, 0]
        for m in range(1, N_NODES):
            acc = acc + xs[m] * adj_ref[n, m]
        o_ref[:, n, :] = jnp.dot(acc, w, preferred_element_type=jnp.float32) + b


def kernel(input, adj, weight, bias):
    bias2d = bias.reshape(1, OUT_F)
    grid = (BATCH // BM,)
    return pl.pallas_call(
        _gcn_tile,
        grid=grid,
        in_specs=[
            pl.BlockSpec(memory_space=pltpu.SMEM),
            pl.BlockSpec((BM, N_NODES, IN_F), lambda i: (i, 0, 0)),
            pl.BlockSpec((IN_F, OUT_F), lambda i: (0, 0)),
            pl.BlockSpec((1, OUT_F), lambda i: (0, 0)),
        ],
        out_specs=pl.BlockSpec((BM, N_NODES, OUT_F), lambda i: (i, 0, 0)),
        out_shape=jax.ShapeDtypeStruct((BATCH, N_NODES, OUT_F), jnp.float32),
        compiler_params=pltpu.CompilerParams(
            dimension_semantics=("parallel",),
        ),
    )(adj, input, weight, bias2d)


# speedup vs baseline: 2.7407x; 2.7407x over previous
import jax
import jax.numpy as jnp
from jax.experimental import pallas as pl
from jax.experimental.pallas import tpu as pltpu

BATCH = 4096
N_NODES = 8
IN_F = 256
OUT_F = 256
BM = 256


def _gcn_tile(adj_ref, *refs):
    xs = refs[:N_NODES]
    w_ref, b_ref, o_ref = refs[N_NODES:]
    n = pl.program_id(1)
    acc = xs[0][...] * adj_ref[n, 0]
    for m in range(1, N_NODES):
        acc = acc + xs[m][...] * adj_ref[n, m]
    out = jnp.dot(
        acc.astype(jnp.bfloat16), w_ref[...], preferred_element_type=jnp.float32
    )
    o_ref[...] = out + b_ref[...]


def kernel(input, adj, weight, bias):
    x2 = input.reshape(BATCH, N_NODES * IN_F)
    w_bf = weight.astype(jnp.bfloat16)
    bias2d = bias.reshape(1, OUT_F)
    grid = (BATCH // BM, N_NODES)
    in_specs = [pl.BlockSpec(memory_space=pltpu.SMEM)]
    for m in range(N_NODES):
        in_specs.append(pl.BlockSpec((BM, IN_F), lambda i, n, m=m: (i, m)))
    in_specs.append(pl.BlockSpec((IN_F, OUT_F), lambda i, n: (0, 0)))
    in_specs.append(pl.BlockSpec((1, OUT_F), lambda i, n: (0, 0)))
    out2 = pl.pallas_call(
        _gcn_tile,
        grid=grid,
        in_specs=in_specs,
        out_specs=pl.BlockSpec((BM, OUT_F), lambda i, n: (i, n)),
        out_shape=jax.ShapeDtypeStruct((BATCH, N_NODES * OUT_F), jnp.float32),
        compiler_params=pltpu.CompilerParams(
            dimension_semantics=("parallel", "arbitrary"),
        ),
    )(adj, *([x2] * N_NODES), w_bf, bias2d)
    return out2.reshape(BATCH, N_NODES, OUT_F)


# speedup vs baseline: 4.2990x; 1.5686x over previous
import jax
import jax.numpy as jnp
from jax.experimental import pallas as pl
from jax.experimental.pallas import tpu as pltpu

BATCH = 4096
N_NODES = 8
IN_F = 256
OUT_F = 256
BM = 256


def _gcn_tile(adj_ref, x_ref, w_ref, b_ref, o_ref, xbf_ref):
    for m in range(N_NODES):
        s = slice(m * IN_F, (m + 1) * IN_F)
        xbf_ref[:, s] = x_ref[:, s].astype(jnp.bfloat16)
    w = w_ref[...]
    b = b_ref[...]
    for n in range(N_NODES):
        acc = xbf_ref[:, 0:IN_F] * adj_ref[n, 0].astype(jnp.bfloat16)
        for m in range(1, N_NODES):
            s = slice(m * IN_F, (m + 1) * IN_F)
            acc = acc + xbf_ref[:, s] * adj_ref[n, m].astype(jnp.bfloat16)
        out = jnp.dot(acc, w, preferred_element_type=jnp.float32)
        o_ref[:, n * OUT_F:(n + 1) * OUT_F] = out + b


def kernel(input, adj, weight, bias):
    x2 = input.reshape(BATCH, N_NODES * IN_F)
    w_bf = weight.astype(jnp.bfloat16)
    bias2d = bias.reshape(1, OUT_F)
    grid = (BATCH // BM,)
    out2 = pl.pallas_call(
        _gcn_tile,
        grid=grid,
        in_specs=[
            pl.BlockSpec(memory_space=pltpu.SMEM),
            pl.BlockSpec((BM, N_NODES * IN_F), lambda i: (i, 0)),
            pl.BlockSpec((IN_F, OUT_F), lambda i: (0, 0)),
            pl.BlockSpec((1, OUT_F), lambda i: (0, 0)),
        ],
        out_specs=pl.BlockSpec((BM, N_NODES * OUT_F), lambda i: (i, 0)),
        out_shape=jax.ShapeDtypeStruct((BATCH, N_NODES * OUT_F), jnp.float32),
        scratch_shapes=[pltpu.VMEM((BM, N_NODES * IN_F), jnp.bfloat16)],
        compiler_params=pltpu.CompilerParams(
            dimension_semantics=("parallel",),
        ),
    )(adj, x2, w_bf, bias2d)
    return out2.reshape(BATCH, N_NODES, OUT_F)


# speedup vs baseline: 10.6414x; 2.4753x over previous
import jax
import jax.numpy as jnp
from jax.experimental import pallas as pl
from jax.experimental.pallas import tpu as pltpu

BATCH = 4096
N_NODES = 8
IN_F = 256
OUT_F = 256
BM = 256


def _gcn_tile(x_ref, c_ref, w_ref, b_ref, o_ref):
    x = x_ref[...]
    s = jnp.dot(
        x.astype(jnp.bfloat16), w_ref[...], preferred_element_type=jnp.float32
    )
    s3 = s.reshape(BM, N_NODES, OUT_F)
    acc = s3 * c_ref[0]
    for d in range(1, N_NODES):
        acc = acc + jnp.roll(s3, -d, axis=1) * c_ref[d]
    o_ref[...] = (acc + b_ref[...]).reshape(BM * N_NODES, OUT_F)


def kernel(input, adj, weight, bias):
    x2 = input.reshape(BATCH * N_NODES, IN_F)
    w_bf = weight.astype(jnp.bfloat16)
    bias2d = bias.reshape(1, OUT_F)
    j = jnp.arange(N_NODES)
    coef = adj[j[None, :], (j[None, :] + j[:, None]) % N_NODES]
    coef_planes = jnp.broadcast_to(
        coef[:, :, None], (N_NODES, N_NODES, OUT_F)
    )
    rows = BM * N_NODES
    grid = (BATCH // BM,)
    out2 = pl.pallas_call(
        _gcn_tile,
        grid=grid,
        in_specs=[
            pl.BlockSpec((rows, IN_F), lambda i: (i, 0)),
            pl.BlockSpec((N_NODES, N_NODES, OUT_F), lambda i: (0, 0, 0)),
            pl.BlockSpec((IN_F, OUT_F), lambda i: (0, 0)),
            pl.BlockSpec((1, OUT_F), lambda i: (0, 0)),
        ],
        out_specs=pl.BlockSpec((rows, OUT_F), lambda i: (i, 0)),
        out_shape=jax.ShapeDtypeStruct((BATCH * N_NODES, OUT_F), jnp.float32),
        compiler_params=pltpu.CompilerParams(
            dimension_semantics=("parallel",),
        ),
    )(x2, coef_planes, w_bf, bias2d)
    return out2.reshape(BATCH, N_NODES, OUT_F)


# speedup vs baseline: 11.7828x; 1.1073x over previous
import jax
import jax.numpy as jnp
from jax.experimental import pallas as pl
from jax.experimental.pallas import tpu as pltpu

BATCH = 4096
N_NODES = 8
IN_F = 256
OUT_F = 256
BM = 256
CHUNK = 256


def _gcn_tile(x_ref, p_ref, w_ref, b_ref, o_ref):
    x = x_ref[...]
    s = jnp.dot(
        x.astype(jnp.bfloat16), w_ref[...], preferred_element_type=jnp.float32
    )
    sb = s.astype(jnp.bfloat16)
    p = p_ref[...]
    b = b_ref[...]
    for k in range(BM * N_NODES // CHUNK):
        r = slice(k * CHUNK, (k + 1) * CHUNK)
        o_ref[r, :] = (
            jnp.dot(p, sb[r, :], preferred_element_type=jnp.float32) + b
        )


def kernel(input, adj, weight, bias):
    x2 = input.reshape(BATCH * N_NODES, IN_F)
    w_bf = weight.astype(jnp.bfloat16)
    p_bf = jnp.kron(jnp.eye(CHUNK // N_NODES, dtype=adj.dtype), adj).astype(
        jnp.bfloat16
    )
    bias2d = bias.reshape(1, OUT_F)
    rows = BM * N_NODES
    grid = (BATCH // BM,)
    out2 = pl.pallas_call(
        _gcn_tile,
        grid=grid,
        in_specs=[
            pl.BlockSpec((rows, IN_F), lambda i: (i, 0)),
            pl.BlockSpec((CHUNK, CHUNK), lambda i: (0, 0)),
            pl.BlockSpec((IN_F, OUT_F), lambda i: (0, 0)),
            pl.BlockSpec((1, OUT_F), lambda i: (0, 0)),
        ],
        out_specs=pl.BlockSpec((rows, OUT_F), lambda i: (i, 0)),
        out_shape=jax.ShapeDtypeStruct((BATCH * N_NODES, OUT_F), jnp.float32),
        compiler_params=pltpu.CompilerParams(
            dimension_semantics=("parallel",),
        ),
    )(x2, p_bf, w_bf, bias2d)
    return out2.reshape(BATCH, N_NODES, OUT_F)
